# baseline (speedup 1.0000x reference)
import functools

import jax
import jax.numpy as jnp
from jax import lax
from jax.experimental import pallas as pl
from jax.experimental.pallas import tpu as pltpu
from jax.experimental.pallas import tpu_sc as plsc


def _make_sc_gather(V, D, B):
    info = plsc.get_sparse_core_info()
    NC, NS = info.num_cores, info.num_subcores
    NW = NC * NS
    assert B % NW == 0
    b_per_w = B // NW
    CH = next(c for c in (128, 120, 112, 104, 96, 88, 80, 72, 64, 56, 48,
                          40, 32, 24, 16, 8) if b_per_w % c == 0)
    assert b_per_w % CH == 0
    n_ch = b_per_w // CH

    mesh = plsc.VectorSubcoreMesh(core_axis_name="c", subcore_axis_name="s")

    @functools.partial(
        pl.kernel,
        mesh=mesh,
        out_type=jax.ShapeDtypeStruct((B, D), jnp.float32),
        scratch_types=[
            pltpu.VMEM((b_per_w,), jnp.int32),
            pltpu.VMEM((CH, D), jnp.float32),
            pltpu.VMEM((CH, D), jnp.float32),
            pltpu.SemaphoreType.DMA,
            pltpu.SemaphoreType.DMA,
        ],
    )
    def gather(table_hbm, idx_hbm, out_hbm, idx_v, rows_v0, rows_v1, s0, s1):
        wid = lax.axis_index("s") * NC + lax.axis_index("c")
        base = wid * b_per_w
        pltpu.sync_copy(idx_hbm.at[pl.ds(base, b_per_w)], idx_v)

        bufs = (rows_v0, rows_v1)
        sems = (s0, s1)
        cur = pltpu.async_copy(
            table_hbm.at[idx_v.at[pl.ds(0, CH)]], bufs[0], sems[0])
        for j in range(n_ch):
            cur.wait()
            if j + 1 < n_ch:
                cur = pltpu.async_copy(
                    table_hbm.at[idx_v.at[pl.ds((j + 1) * CH, CH)]],
                    bufs[(j + 1) % 2], sems[(j + 1) % 2],
                )
            pltpu.sync_copy(bufs[j % 2], out_hbm.at[pl.ds(base + j * CH, CH)])

    return gather


def _make_mm_body(aliased):
    def _mm_body(*refs):
        if aliased:
            _, emb_ref, w_ref, b_ref, out_ref = refs
        else:
            emb_ref, w_ref, b_ref, out_ref = refs
        out_ref[...] = (
            jnp.dot(emb_ref[...], w_ref[...],
                    preferred_element_type=jnp.float32)
            + b_ref[...]
        )
    return _mm_body


def _project_chunk(out_prev, emb_c, W, b2, blk0, n_blk, BR, B, eblk0=0):
    D = emb_c.shape[1]
    H = W.shape[1]
    specs = [
        pl.BlockSpec((BR, D), lambda i: (eblk0 + i, 0)),
        pl.BlockSpec((D, H), lambda i: (0, 0)),
        pl.BlockSpec((1, H), lambda i: (0, 0)),
    ]
    if out_prev is None:
        return pl.pallas_call(
            _make_mm_body(False),
            grid=(n_blk,),
            in_specs=specs,
            out_specs=pl.BlockSpec((BR, H), lambda i: (blk0 + i, 0)),
            out_shape=jax.ShapeDtypeStruct((B, H), jnp.float32),
        )(emb_c, W, b2)
    return pl.pallas_call(
        _make_mm_body(True),
        grid=(n_blk,),
        in_specs=[pl.BlockSpec(memory_space=pl.ANY)] + specs,
        out_specs=pl.BlockSpec((BR, H), lambda i: (blk0 + i, 0)),
        out_shape=jax.ShapeDtypeStruct((B, H), jnp.float32),
        input_output_aliases={0: 0},
    )(out_prev, emb_c, W, b2)


def kernel(indices, table, W, b):
    Bt, L = indices.shape
    V, D = table.shape
    H = W.shape[1]
    B = Bt * L

    idx_t = jnp.swapaxes(indices, 0, 1).reshape(-1).astype(jnp.int32)

    N_CHUNKS = 4
    CB = B // N_CHUNKS
    BR = 3200
    assert CB % BR == 0

    gather = _make_sc_gather(V, D, CB)
    embs = [
        gather(table, lax.dynamic_slice_in_dim(idx_t, c * CB, CB))
        for c in range(N_CHUNKS)
    ]

    out = None
    b2 = b.reshape(1, H)
    n_blk = CB // BR
    for c in range(N_CHUNKS):
        out = _project_chunk(out, embs[c], W, b2,
                             blk0=c * n_blk, n_blk=n_blk, BR=BR, B=B)

    return jnp.transpose(out.reshape(L, Bt, H), (1, 0, 2))


# speedup vs baseline: 1.0135x; 1.0135x over previous
import functools

import jax
import jax.numpy as jnp
from jax import lax
from jax.experimental import pallas as pl
from jax.experimental.pallas import tpu as pltpu
from jax.experimental.pallas import tpu_sc as plsc


def _make_sc_gather(V, D, B):
    info = plsc.get_sparse_core_info()
    NC, NS = info.num_cores, info.num_subcores
    NW = NC * NS
    assert B % NW == 0
    b_per_w = B // NW
    CH = next(c for c in (128, 120, 112, 104, 96, 88, 80, 72, 64, 56, 48,
                          40, 32, 24, 16, 8) if b_per_w % c == 0)
    assert b_per_w % CH == 0
    n_ch = b_per_w // CH

    mesh = plsc.VectorSubcoreMesh(core_axis_name="c", subcore_axis_name="s")

    @functools.partial(
        pl.kernel,
        mesh=mesh,
        out_type=jax.ShapeDtypeStruct((B, D), jnp.float32),
        scratch_types=[
            pltpu.VMEM((b_per_w,), jnp.int32),
            pltpu.VMEM((CH, D), jnp.float32),
            pltpu.VMEM((CH, D), jnp.float32),
            pltpu.SemaphoreType.DMA,
            pltpu.SemaphoreType.DMA,
        ],
    )
    def gather(table_hbm, idx_hbm, out_hbm, idx_v, rows_v0, rows_v1, s0, s1):
        wid = lax.axis_index("s") * NC + lax.axis_index("c")
        base = wid * b_per_w
        pltpu.sync_copy(idx_hbm.at[pl.ds(base, b_per_w)], idx_v)

        bufs = (rows_v0, rows_v1)
        sems = (s0, s1)
        cur = pltpu.async_copy(
            table_hbm.at[idx_v.at[pl.ds(0, CH)]], bufs[0], sems[0])
        for j in range(n_ch):
            cur.wait()
            if j + 1 < n_ch:
                cur = pltpu.async_copy(
                    table_hbm.at[idx_v.at[pl.ds((j + 1) * CH, CH)]],
                    bufs[(j + 1) % 2], sems[(j + 1) % 2],
                )
            pltpu.sync_copy(bufs[j % 2], out_hbm.at[pl.ds(base + j * CH, CH)])

    return gather


def _make_mm_body(aliased):
    def _mm_body(*refs):
        if aliased:
            _, emb_ref, w_ref, b_ref, out_ref = refs
        else:
            emb_ref, w_ref, b_ref, out_ref = refs
        out_ref[...] = (
            jnp.dot(emb_ref[...], w_ref[...],
                    preferred_element_type=jnp.float32)
            + b_ref[...]
        )
    return _mm_body


def _project_chunk(out_prev, emb_c, W, b2, blk0, n_blk, BR, B, eblk0=0):
    D = emb_c.shape[1]
    H = W.shape[1]
    specs = [
        pl.BlockSpec((BR, D), lambda i: (eblk0 + i, 0)),
        pl.BlockSpec((D, H), lambda i: (0, 0)),
        pl.BlockSpec((1, H), lambda i: (0, 0)),
    ]
    if out_prev is None:
        return pl.pallas_call(
            _make_mm_body(False),
            grid=(n_blk,),
            in_specs=specs,
            out_specs=pl.BlockSpec((BR, H), lambda i: (blk0 + i, 0)),
            out_shape=jax.ShapeDtypeStruct((B, H), jnp.float32),
        )(emb_c, W, b2)
    return pl.pallas_call(
        _make_mm_body(True),
        grid=(n_blk,),
        in_specs=[pl.BlockSpec(memory_space=pl.ANY)] + specs,
        out_specs=pl.BlockSpec((BR, H), lambda i: (blk0 + i, 0)),
        out_shape=jax.ShapeDtypeStruct((B, H), jnp.float32),
        input_output_aliases={0: 0},
    )(out_prev, emb_c, W, b2)


def kernel(indices, table, W, b):
    Bt, L = indices.shape
    V, D = table.shape
    H = W.shape[1]
    B = Bt * L

    idx_t = jnp.swapaxes(indices, 0, 1).reshape(-1).astype(jnp.int32)

    sizes = (B // 8, B // 8, B // 4, B // 2)
    brs = (1600, 1600, 2560, 2560)

    embs = []
    off = 0
    for cb in sizes:
        embs.append(_make_sc_gather(V, D, cb)(
            table, lax.dynamic_slice_in_dim(idx_t, off, cb)))
        off += cb

    out = None
    b2 = b.reshape(1, H)
    off = 0
    for cb, br, emb_c in zip(sizes, brs, embs):
        assert cb % br == 0 and off % br == 0
        out = _project_chunk(out, emb_c, W, b2,
                             blk0=off // br, n_blk=cb // br, BR=br, B=B)
        off += cb

    return jnp.transpose(out.reshape(L, Bt, H), (1, 0, 2))


# speedup vs baseline: 1.0233x; 1.0097x over previous
import functools

import jax
import jax.numpy as jnp
from jax import lax
from jax.experimental import pallas as pl
from jax.experimental.pallas import tpu as pltpu
from jax.experimental.pallas import tpu_sc as plsc


def _make_sc_gather(V, D, B):
    info = plsc.get_sparse_core_info()
    NC, NS = info.num_cores, info.num_subcores
    NW = NC * NS
    assert B % NW == 0
    b_per_w = B // NW
    CH = next(c for c in (128, 120, 112, 104, 96, 88, 80, 72, 64, 56, 48,
                          40, 32, 24, 16, 8) if b_per_w % c == 0)
    assert b_per_w % CH == 0
    n_ch = b_per_w // CH

    mesh = plsc.VectorSubcoreMesh(core_axis_name="c", subcore_axis_name="s")

    @functools.partial(
        pl.kernel,
        mesh=mesh,
        out_type=jax.ShapeDtypeStruct((B, D), jnp.float32),
        scratch_types=[
            pltpu.VMEM((b_per_w,), jnp.int32),
            pltpu.VMEM((CH, D), jnp.float32),
            pltpu.VMEM((CH, D), jnp.float32),
            pltpu.SemaphoreType.DMA,
            pltpu.SemaphoreType.DMA,
        ],
    )
    def gather(table_hbm, idx_hbm, out_hbm, idx_v, rows_v0, rows_v1, s0, s1):
        wid = lax.axis_index("s") * NC + lax.axis_index("c")
        base = wid * b_per_w
        pltpu.sync_copy(idx_hbm.at[pl.ds(base, b_per_w)], idx_v)

        bufs = (rows_v0, rows_v1)
        sems = (s0, s1)
        cur = pltpu.async_copy(
            table_hbm.at[idx_v.at[pl.ds(0, CH)]], bufs[0], sems[0])
        for j in range(n_ch):
            cur.wait()
            if j + 1 < n_ch:
                cur = pltpu.async_copy(
                    table_hbm.at[idx_v.at[pl.ds((j + 1) * CH, CH)]],
                    bufs[(j + 1) % 2], sems[(j + 1) % 2],
                )
            pltpu.sync_copy(bufs[j % 2], out_hbm.at[pl.ds(base + j * CH, CH)])

    return gather


def _make_mm_body(aliased):
    def _mm_body(*refs):
        if aliased:
            _, emb_ref, w_ref, b_ref, out_ref = refs
        else:
            emb_ref, w_ref, b_ref, out_ref = refs
        out_ref[...] = (
            jnp.dot(emb_ref[...], w_ref[...],
                    preferred_element_type=jnp.float32)
            + b_ref[...]
        )
    return _mm_body


def _project_chunk(out_prev, emb_c, W, b2, blk0, n_blk, BR, B, eblk0=0):
    D = emb_c.shape[1]
    H = W.shape[1]
    specs = [
        pl.BlockSpec((BR, D), lambda i: (eblk0 + i, 0)),
        pl.BlockSpec((D, H), lambda i: (0, 0)),
        pl.BlockSpec((1, H), lambda i: (0, 0)),
    ]
    if out_prev is None:
        return pl.pallas_call(
            _make_mm_body(False),
            grid=(n_blk,),
            in_specs=specs,
            out_specs=pl.BlockSpec((BR, H), lambda i: (blk0 + i, 0)),
            out_shape=jax.ShapeDtypeStruct((B, H), jnp.float32),
        )(emb_c, W, b2)
    return pl.pallas_call(
        _make_mm_body(True),
        grid=(n_blk,),
        in_specs=[pl.BlockSpec(memory_space=pl.ANY)] + specs,
        out_specs=pl.BlockSpec((BR, H), lambda i: (blk0 + i, 0)),
        out_shape=jax.ShapeDtypeStruct((B, H), jnp.float32),
        input_output_aliases={0: 0},
    )(out_prev, emb_c, W, b2)


def kernel(indices, table, W, b):
    Bt, L = indices.shape
    V, D = table.shape
    H = W.shape[1]
    B = Bt * L

    idx_t = jnp.swapaxes(indices, 0, 1).reshape(-1).astype(jnp.int32)

    sizes = (B // 4, B // 4, B // 4, B // 4)
    brs = (2560, 2560, 2560, 2560)

    embs = []
    off = 0
    for cb in sizes:
        embs.append(_make_sc_gather(V, D, cb)(
            table, lax.dynamic_slice_in_dim(idx_t, off, cb)))
        off += cb

    out = None
    b2 = b.reshape(1, H)
    off = 0
    for cb, br, emb_c in zip(sizes, brs, embs):
        assert cb % br == 0 and off % br == 0
        out = _project_chunk(out, emb_c, W, b2,
                             blk0=off // br, n_blk=cb // br, BR=br, B=B)
        off += cb

    return jnp.transpose(out.reshape(L, Bt, H), (1, 0, 2))
